# Initial kernel scaffold; baseline (speedup 1.0000x reference)
import jax
import jax.numpy as jnp
from jax.experimental import pallas as pl


def kernel(item, cat, W_item, W_cat):
    raise NotImplementedError("write your pallas kernel here")


# speedup vs baseline: 1.8487x; 1.8487x over previous
import functools

import jax
import jax.numpy as jnp
from jax import lax
from jax.experimental import pallas as pl
from jax.experimental.pallas import tpu as pltpu
from jax.experimental.pallas import tpu_sc as plsc

_NC = 2
_NS = 16
_NW = _NC * _NS


@functools.lru_cache(maxsize=None)
def _build(B, L, V_item, D_item, V_cat, D_cat):
    D = D_item + D_cat
    b_per_w = B // _NW
    LP = ((L + 15) // 16) * 16
    n_full = (L // 16)
    rem = L - n_full * 16

    mesh = plsc.VectorSubcoreMesh(
        core_axis_name="c", subcore_axis_name="s",
        num_cores=_NC, num_subcores=_NS)

    @functools.partial(
        pl.kernel,
        mesh=mesh,
        compiler_params=pltpu.CompilerParams(
            needs_layout_passes=False, use_tc_tiling_on_sc=False),
        out_type=jax.ShapeDtypeStruct((B, D, L), jnp.float32),
        scratch_types=[
            pltpu.VMEM((L,), jnp.int32),
            pltpu.VMEM((L,), jnp.int32),
            pltpu.VMEM((LP, D_item), jnp.float32),
            pltpu.VMEM((LP, D_cat), jnp.float32),
            pltpu.VMEM((D, L), jnp.float32),
            pltpu.SemaphoreType.DMA,
            pltpu.SemaphoreType.DMA,
        ],
    )
    def seq_embed(item_h, cat_h, wi_h, wc_h, out_h,
                  idx_i, idx_c, rows_i, rows_c, obuf, sem_i, sem_c):
        wid = lax.axis_index("s") * _NC + lax.axis_index("c")
        lanes = lax.iota(jnp.int32, 16)

        def per_row(j, carry):
            b = wid * b_per_w + j
            pltpu.sync_copy(item_h.at[b], idx_i)
            pltpu.sync_copy(cat_h.at[b], idx_c)
            cp_i = pltpu.async_copy(wi_h.at[idx_i], rows_i.at[pl.ds(0, L)],
                                    sem_i)
            cp_c = pltpu.async_copy(wc_h.at[idx_c], rows_c.at[pl.ds(0, L)],
                                    sem_c)
            cp_i.wait()
            cp_c.wait()

            def transpose_col(rows, dbase, d, carry):
                col = jnp.full((16,), d, jnp.int32)
                for r in range(0, n_full * 16, 16):
                    vals = plsc.load_gather(rows, [lanes + r, col])
                    obuf[dbase + d, pl.ds(r, 16)] = vals
                if rem:
                    r = n_full * 16
                    vals = plsc.load_gather(rows, [lanes + r, col])
                    plsc.store_scatter(
                        obuf,
                        [jnp.full((16,), dbase + d, jnp.int32), lanes + r],
                        vals, mask=lanes < rem)
                return carry

            lax.fori_loop(0, D_item,
                          functools.partial(transpose_col, rows_i, 0), 0)
            lax.fori_loop(0, D_cat,
                          functools.partial(transpose_col, rows_c, D_item), 0)
            pltpu.sync_copy(obuf, out_h.at[b])
            return carry

        lax.fori_loop(0, b_per_w, per_row, 0)

    return seq_embed


def kernel(item, cat, W_item, W_cat):
    B, L = item.shape
    V_item, D_item = W_item.shape
    V_cat, D_cat = W_cat.shape
    fn = _build(B, L, V_item, D_item, V_cat, D_cat)
    return fn(item.astype(jnp.int32), cat.astype(jnp.int32), W_item, W_cat)


# speedup vs baseline: 3.0699x; 1.6605x over previous
import functools

import jax
import jax.numpy as jnp
from jax import lax
from jax.experimental import pallas as pl
from jax.experimental.pallas import tpu as pltpu
from jax.experimental.pallas import tpu_sc as plsc

_NC = 2
_NS = 16
_NW = _NC * _NS


@functools.lru_cache(maxsize=None)
def _build(B, L, V_item, D_item, V_cat, D_cat):
    D = D_item + D_cat
    b_per_w = B // _NW
    LP = ((L + 15) // 16) * 16
    n_full = L // 16
    rem = L - n_full * 16

    mesh = plsc.VectorSubcoreMesh(
        core_axis_name="c", subcore_axis_name="s",
        num_cores=_NC, num_subcores=_NS)

    @functools.partial(
        pl.kernel,
        mesh=mesh,
        compiler_params=pltpu.CompilerParams(
            needs_layout_passes=False, use_tc_tiling_on_sc=False),
        out_type=jax.ShapeDtypeStruct((B, D, L), jnp.float32),
        scratch_types=[
            pltpu.VMEM((2, L), jnp.int32),
            pltpu.VMEM((2, L), jnp.int32),
            pltpu.VMEM((2, LP, D_item), jnp.float32),
            pltpu.VMEM((2, LP, D_cat), jnp.float32),
            pltpu.VMEM((2, D, L), jnp.float32),
            pltpu.SemaphoreType.DMA((2,)),
            pltpu.SemaphoreType.DMA((2,)),
            pltpu.SemaphoreType.DMA((2,)),
            pltpu.SemaphoreType.DMA((2,)),
            pltpu.SemaphoreType.DMA((2,)),
        ],
    )
    def seq_embed(item_h, cat_h, wi_h, wc_h, out_h,
                  idx_i, idx_c, rows_i, rows_c, obuf,
                  sxi, sxc, sgi, sgc, sob):
        wid = lax.axis_index("s") * _NC + lax.axis_index("c")
        b0 = wid * b_per_w
        lanes = lax.iota(jnp.int32, 16)

        def mk_idx_i(slot, brow):
            return pltpu.make_async_copy(
                item_h.at[brow], idx_i.at[slot], sxi.at[slot])

        def mk_idx_c(slot, brow):
            return pltpu.make_async_copy(
                cat_h.at[brow], idx_c.at[slot], sxc.at[slot])

        def mk_gi(slot):
            return pltpu.make_async_copy(
                wi_h.at[idx_i.at[slot]],
                rows_i.at[slot, pl.ds(0, L)], sgi.at[slot])

        def mk_gc(slot):
            return pltpu.make_async_copy(
                wc_h.at[idx_c.at[slot]],
                rows_c.at[slot, pl.ds(0, L)], sgc.at[slot])

        def mk_ob(slot, brow):
            return pltpu.make_async_copy(
                obuf.at[slot], out_h.at[brow], sob.at[slot])

        def transpose_block(rows, ob, dbase, dsize):
            @plsc.parallel_loop(0, dsize, 1, unroll=2)
            def _(d):
                col = jnp.full((16,), d, jnp.int32)
                for r in range(0, n_full * 16, 16):
                    ob[dbase + d, pl.ds(r, 16)] = plsc.load_gather(
                        rows, [lanes + r, col])
                if rem:
                    r = n_full * 16
                    vals = plsc.load_gather(rows, [lanes + r, col])
                    plsc.store_scatter(
                        ob, [jnp.full((16,), dbase + d, jnp.int32),
                             lanes + r],
                        vals, mask=lanes < rem)

        pltpu.sync_copy(item_h.at[b0], idx_i.at[0])
        pltpu.sync_copy(cat_h.at[b0], idx_c.at[0])
        mk_gi(0).start()
        mk_gc(0).start()
        mk_idx_i(1, b0 + 1).start()
        mk_idx_c(1, b0 + 1).start()

        def iter_body(jj, carry):
            for p in (0, 1):
                nxt = 1 - p
                j = jj * 2 + p
                b = b0 + j
                mk_gi(p).wait()
                mk_gc(p).wait()

                @pl.when(j + 2 < b_per_w)
                def _():
                    mk_idx_i(p, b + 2).start()
                    mk_idx_c(p, b + 2).start()

                def issue_next():
                    mk_idx_i(nxt, b).wait()
                    mk_idx_c(nxt, b).wait()
                    mk_gi(nxt).start()
                    mk_gc(nxt).start()

                if p == 0:
                    issue_next()
                else:
                    pl.when(j + 1 < b_per_w)(issue_next)

                @pl.when(j >= 2)
                def _():
                    mk_ob(p, b).wait()

                transpose_block(rows_i.at[p], obuf.at[p], 0, D_item)
                transpose_block(rows_c.at[p], obuf.at[p], D_item, D_cat)
                mk_ob(p, b).start()
            return carry

        lax.fori_loop(0, b_per_w // 2, iter_body, 0)
        mk_ob(0, b0).wait()
        mk_ob(1, b0).wait()

    return seq_embed


def kernel(item, cat, W_item, W_cat):
    B, L = item.shape
    V_item, D_item = W_item.shape
    V_cat, D_cat = W_cat.shape
    fn = _build(B, L, V_item, D_item, V_cat, D_cat)
    return fn(item.astype(jnp.int32), cat.astype(jnp.int32), W_item, W_cat)


# speedup vs baseline: 4.5501x; 1.4822x over previous
import functools

import jax
import jax.numpy as jnp
from jax import lax
from jax.experimental import pallas as pl
from jax.experimental.pallas import tpu as pltpu
from jax.experimental.pallas import tpu_sc as plsc

_NC = 2
_NS = 16
_NW = _NC * _NS


@functools.lru_cache(maxsize=None)
def _build(B, L, V_item, D_item, V_cat, D_cat):
    D = D_item + D_cat
    b_per_w = B // _NW
    LQ = L + 1

    mesh = plsc.VectorSubcoreMesh(
        core_axis_name="c", subcore_axis_name="s",
        num_cores=_NC, num_subcores=_NS)

    @functools.partial(
        pl.kernel,
        mesh=mesh,
        compiler_params=pltpu.CompilerParams(
            needs_layout_passes=False, use_tc_tiling_on_sc=False),
        out_type=jax.ShapeDtypeStruct((B, D, L), jnp.float32),
        scratch_types=[
            pltpu.VMEM((2, L), jnp.int32),
            pltpu.VMEM((2, L), jnp.int32),
            pltpu.VMEM((2, L, D_item), jnp.float32),
            pltpu.VMEM((2, L, D_cat), jnp.float32),
            pltpu.VMEM((2, D, LQ), jnp.float32),
            pltpu.SemaphoreType.DMA((2,)),
            pltpu.SemaphoreType.DMA((2,)),
            pltpu.SemaphoreType.DMA((2,)),
            pltpu.SemaphoreType.DMA((2,)),
            pltpu.SemaphoreType.DMA((2,)),
        ],
    )
    def seq_embed(item_h, cat_h, wi_h, wc_h, out_h,
                  idx_i, idx_c, rows_i, rows_c, obuf,
                  sxi, sxc, sgi, sgc, sob):
        wid = lax.axis_index("s") * _NC + lax.axis_index("c")
        b0 = wid * b_per_w
        lanes = lax.iota(jnp.int32, 16)

        def mk_idx_i(slot, brow):
            return pltpu.make_async_copy(
                item_h.at[brow], idx_i.at[slot], sxi.at[slot])

        def mk_idx_c(slot, brow):
            return pltpu.make_async_copy(
                cat_h.at[brow], idx_c.at[slot], sxc.at[slot])

        def mk_gi(slot):
            return pltpu.make_async_copy(
                wi_h.at[idx_i.at[slot]],
                rows_i.at[slot], sgi.at[slot])

        def mk_gc(slot):
            return pltpu.make_async_copy(
                wc_h.at[idx_c.at[slot]],
                rows_c.at[slot], sgc.at[slot])

        def mk_ob(slot, brow):
            return pltpu.make_async_copy(
                obuf.at[slot, :, pl.ds(0, L)], out_h.at[brow],
                sob.at[slot])

        def transpose_block(rows, ob, dbase, dsize):
            @plsc.parallel_loop(0, L, 1, unroll=2)
            def _(l):
                coll = jnp.full((16,), l, jnp.int32)
                for ci in range(dsize // 16):
                    vals = rows[l, pl.ds(ci * 16, 16)]
                    plsc.store_scatter(
                        ob, [lanes + (dbase + ci * 16), coll], vals)

        pltpu.sync_copy(item_h.at[b0], idx_i.at[0])
        pltpu.sync_copy(cat_h.at[b0], idx_c.at[0])
        mk_gi(0).start()
        mk_gc(0).start()
        mk_idx_i(1, b0 + 1).start()
        mk_idx_c(1, b0 + 1).start()

        def iter_body(jj, carry):
            for p in (0, 1):
                nxt = 1 - p
                j = jj * 2 + p
                b = b0 + j
                mk_gi(p).wait()
                mk_gc(p).wait()

                @pl.when(j + 2 < b_per_w)
                def _():
                    mk_idx_i(p, b + 2).start()
                    mk_idx_c(p, b + 2).start()

                def issue_next():
                    mk_idx_i(nxt, b).wait()
                    mk_idx_c(nxt, b).wait()
                    mk_gi(nxt).start()
                    mk_gc(nxt).start()

                if p == 0:
                    issue_next()
                else:
                    pl.when(j + 1 < b_per_w)(issue_next)

                @pl.when(j >= 2)
                def _():
                    mk_ob(p, b).wait()

                transpose_block(rows_i.at[p], obuf.at[p], 0, D_item)
                transpose_block(rows_c.at[p], obuf.at[p], D_item, D_cat)
                mk_ob(p, b).start()
            return carry

        lax.fori_loop(0, b_per_w // 2, iter_body, 0)
        mk_ob(0, b0).wait()
        mk_ob(1, b0).wait()

    return seq_embed


def kernel(item, cat, W_item, W_cat):
    B, L = item.shape
    V_item, D_item = W_item.shape
    V_cat, D_cat = W_cat.shape
    fn = _build(B, L, V_item, D_item, V_cat, D_cat)
    return fn(item.astype(jnp.int32), cat.astype(jnp.int32), W_item, W_cat)


# speedup vs baseline: 4.7063x; 1.0343x over previous
import functools

import jax
import jax.numpy as jnp
from jax import lax
from jax.experimental import pallas as pl
from jax.experimental.pallas import tpu as pltpu
from jax.experimental.pallas import tpu_sc as plsc

_NC = 2
_NS = 16
_NW = _NC * _NS


@functools.lru_cache(maxsize=None)
def _build(B, L, V_item, D_item, V_cat, D_cat):
    D = D_item + D_cat
    b_per_w = B // _NW
    LP = ((L + 15) // 16) * 16
    n_full = L // 16
    rem = L - n_full * 16
    TP = LP + 1 if (LP + 1) % 2 else LP + 3
    PC = D_cat + 1
    CHUNK = 125

    mesh = plsc.VectorSubcoreMesh(
        core_axis_name="c", subcore_axis_name="s",
        num_cores=_NC, num_subcores=_NS)

    @functools.partial(
        pl.kernel,
        mesh=mesh,
        compiler_params=pltpu.CompilerParams(
            needs_layout_passes=False, use_tc_tiling_on_sc=False),
        out_type=jax.ShapeDtypeStruct((B, D, L), jnp.float32),
        scratch_types=[
            pltpu.VMEM((2, LP), jnp.int32),
            pltpu.VMEM((2, LP), jnp.int32),
            pltpu.VMEM((2, L, D_item), jnp.float32),
            pltpu.VMEM((D_item, TP), jnp.float32),
            pltpu.VMEM((V_cat, PC), jnp.float32),
            pltpu.VMEM((CHUNK, D_cat), jnp.float32),
            pltpu.VMEM((2, D, L), jnp.float32),
            pltpu.SemaphoreType.DMA((2,)),
            pltpu.SemaphoreType.DMA((2,)),
            pltpu.SemaphoreType.DMA((2,)),
            pltpu.SemaphoreType.DMA((2,)),
        ],
    )
    def seq_embed(item_h, cat_h, wi_h, wc_h, out_h,
                  idx_i, idx_c, rows_i, tmp, wcp, stage, obuf,
                  sxi, sxc, sgi, sob):
        wid = lax.axis_index("s") * _NC + lax.axis_index("c")
        b0 = wid * b_per_w
        lanes = lax.iota(jnp.int32, 16)

        def mk_idx_i(slot, brow):
            return pltpu.make_async_copy(
                item_h.at[brow], idx_i.at[slot, pl.ds(0, L)], sxi.at[slot])

        def mk_idx_c(slot, brow):
            return pltpu.make_async_copy(
                cat_h.at[brow], idx_c.at[slot, pl.ds(0, L)], sxc.at[slot])

        def mk_gi(slot):
            return pltpu.make_async_copy(
                wi_h.at[idx_i.at[slot, pl.ds(0, L)]],
                rows_i.at[slot], sgi.at[slot])

        def mk_ob(slot, brow):
            return pltpu.make_async_copy(
                obuf.at[slot], out_h.at[brow], sob.at[slot])

        for t in range(V_cat // CHUNK):
            pltpu.sync_copy(wc_h.at[pl.ds(t * CHUNK, CHUNK)], stage)

            @plsc.parallel_loop(0, CHUNK, 1, unroll=2)
            def _(r):
                for c in range(D_cat // 16):
                    wcp[t * CHUNK + r, pl.ds(c * 16, 16)] = (
                        stage[r, pl.ds(c * 16, 16)])

        pltpu.sync_copy(item_h.at[b0], idx_i.at[0, pl.ds(0, L)])
        pltpu.sync_copy(cat_h.at[b0], idx_c.at[0, pl.ds(0, L)])
        mk_gi(0).start()
        mk_idx_i(1, b0 + 1).start()
        mk_idx_c(1, b0 + 1).start()

        def iter_body(jj, carry):
            for p in (0, 1):
                nxt = 1 - p
                j = jj * 2 + p
                b = b0 + j
                mk_gi(p).wait()

                @pl.when(j + 2 < b_per_w)
                def _():
                    mk_idx_i(p, b + 2).start()

                def issue_next():
                    mk_idx_i(nxt, b).wait()
                    mk_idx_c(nxt, b).wait()
                    mk_gi(nxt).start()

                if p == 0:
                    issue_next()
                else:
                    pl.when(j + 1 < b_per_w)(issue_next)

                @pl.when(j >= 2)
                def _():
                    mk_ob(p, b).wait()

                @plsc.parallel_loop(0, L, 1, unroll=2)
                def _(l):
                    coll = jnp.full((16,), l, jnp.int32)
                    for ci in range(D_item // 16):
                        plsc.store_scatter(
                            tmp, [lanes + ci * 16, coll],
                            rows_i[p, l, pl.ds(ci * 16, 16)])

                @plsc.parallel_loop(0, D_item, 1, unroll=2)
                def _(d):
                    for r in range(0, n_full * 16, 16):
                        obuf[p, d, pl.ds(r, 16)] = tmp[d, pl.ds(r, 16)]
                    if rem:
                        r = n_full * 16
                        plsc.store_scatter(
                            obuf.at[p], [jnp.full((16,), d, jnp.int32),
                                         lanes + r],
                            tmp[d, pl.ds(r, 16)], mask=lanes < rem)

                def cat_chunk(lc, full_mask):
                    idxv = idx_c[p, pl.ds(lc * 16, 16)]
                    msk = None if full_mask else lanes < rem
                    for d in range(D_cat):
                        vals = plsc.load_gather(
                            wcp, [idxv, jnp.full((16,), d, jnp.int32)],
                            mask=msk)
                        if full_mask:
                            obuf[p, D_item + d, pl.ds(lc * 16, 16)] = vals
                        else:
                            plsc.store_scatter(
                                obuf.at[p],
                                [jnp.full((16,), D_item + d, jnp.int32),
                                 lc * 16 + lanes],
                                vals, mask=msk)

                @plsc.parallel_loop(0, n_full, 1, unroll=1)
                def _(lc):
                    cat_chunk(lc, True)
                if rem:
                    cat_chunk(n_full, False)

                @pl.when(j + 2 < b_per_w)
                def _():
                    mk_idx_c(p, b + 2).start()

                mk_ob(p, b).start()
            return carry

        lax.fori_loop(0, b_per_w // 2, iter_body, 0)
        mk_ob(0, b0).wait()
        mk_ob(1, b0).wait()

    return seq_embed


def kernel(item, cat, W_item, W_cat):
    B, L = item.shape
    V_item, D_item = W_item.shape
    V_cat, D_cat = W_cat.shape
    fn = _build(B, L, V_item, D_item, V_cat, D_cat)
    return fn(item.astype(jnp.int32), cat.astype(jnp.int32), W_item, W_cat)
